# baseline (speedup 1.0000x reference)
import functools

import jax
import jax.numpy as jnp
from jax import lax
from jax.experimental import pallas as pl
from jax.experimental.pallas import tpu as pltpu
from jax.experimental.pallas import tpu_sc as plsc

N = 10000
D = 128
E = 320000
NC = 2
NS = 16
K = 64
NBUF = 4
CHUNKS = 324
C0 = 460
C1 = 2 * CHUNKS - C0
IPS = K * CHUNKS
IPC = IPS * NS
IP = IPC * NC
OUT_ROWS = 10240
ZROWS = 32
WB = OUT_ROWS // NS


def _sc_kernel(x, pair, gl, dl, out, out_sh, pair_sh, ibufs, xbufs, zbuf,
               gsems, ssems):
    c = lax.axis_index("c")
    s = lax.axis_index("s")

    z = jnp.zeros((16,), jnp.float32)

    def zero_row(i, carry):
        for k in range(D // 16):
            zbuf[i, pl.ds(k * 16, 16)] = z
        return carry

    lax.fori_loop(0, ZROWS, zero_row, 0)

    slab = s * WB

    def zero_slab(j, carry):
        pltpu.sync_copy(zbuf, out_sh.at[pl.ds(slab + j * ZROWS, ZROWS)])
        return carry

    lax.fori_loop(0, WB // ZROWS, zero_slab, 0)

    pltpu.sync_copy(pair.at[pl.ds(s * 64, 64)], pair_sh.at[pl.ds(s * 64, 64)])

    plsc.subcore_barrier()

    nch = jnp.where(c == 0, C0, C1)
    first = jnp.where(c == 0, s * (C0 * K), NS * (C0 * K) + s * (C1 * K))

    def load_idx(g, b):
        base = first + g * K
        pltpu.sync_copy(gl.at[pl.ds(base, K)], ibufs.at[b, 0])
        pltpu.sync_copy(dl.at[pl.ds(base, K)], ibufs.at[b, 1])

    def gather(b):
        srcref = x if b % 2 == 0 else pair_sh
        pltpu.async_copy(srcref.at[ibufs.at[b, 0]], xbufs.at[b], gsems.at[b])

    def wait_gather(b):
        srcref = x if b % 2 == 0 else pair_sh
        pltpu.make_async_copy(
            srcref.at[ibufs.at[b, 0]], xbufs.at[b], gsems.at[b]).wait()

    def scatter(b):
        pltpu.async_copy(
            xbufs.at[b], out_sh.at[ibufs.at[b, 1]], ssems.at[b], add=True)

    def wait_scatter(b):
        pltpu.make_async_copy(
            xbufs.at[b], out_sh.at[ibufs.at[b, 1]], ssems.at[b]).wait()

    for b in range(NBUF - 1):
        load_idx(b, b)
        gather(b)

    def step(t, carry):
        for b in range(NBUF):
            g = NBUF * t + b
            wait_gather(b)
            scatter(b)
            nb = (b + NBUF - 1) % NBUF

            @pl.when(g >= 1)
            def _():
                wait_scatter(nb)

            @pl.when(g + NBUF - 1 < nch)
            def _():
                load_idx(g + NBUF - 1, nb)
                gather(nb)
        return carry

    lax.fori_loop(0, nch // NBUF, step, 0)
    wait_scatter(NBUF - 1)

    plsc.subcore_barrier()

    pltpu.sync_copy(out_sh.at[pl.ds(slab, WB)], out.at[c].at[pl.ds(slab, WB)])


def _merge_kernel(p_ref, o_ref):
    o_ref[...] = p_ref[0] + p_ref[1]


@jax.jit
def _propagate(x, pair, gl, dl):
    mesh = plsc.VectorSubcoreMesh(core_axis_name="c", subcore_axis_name="s")
    partials = pl.kernel(
        _sc_kernel,
        out_type=jax.ShapeDtypeStruct((NC, OUT_ROWS, D), jnp.float32),
        mesh=mesh,
        scratch_types=[
            pltpu.VMEM_SHARED((OUT_ROWS, D), jnp.float32),
            pltpu.VMEM_SHARED((1024, D), jnp.float32),
            pltpu.VMEM((NBUF, 2, K), jnp.int32),
            pltpu.VMEM((NBUF, K, D), jnp.float32),
            pltpu.VMEM((ZROWS, D), jnp.float32),
            pltpu.SemaphoreType.DMA((NBUF,)),
            pltpu.SemaphoreType.DMA((NBUF,)),
        ],
    )(x, pair, gl, dl)
    blk = 1024
    merged = pl.pallas_call(
        _merge_kernel,
        grid=(OUT_ROWS // blk,),
        in_specs=[pl.BlockSpec((NC, blk, D), lambda i: (0, i, 0))],
        out_specs=pl.BlockSpec((blk, D), lambda i: (i, 0)),
        out_shape=jax.ShapeDtypeStruct((OUT_ROWS, D), jnp.float32),
    )(partials)
    return merged


def kernel(x, e_feat, e_attr, emb_table):
    src = e_feat[0].astype(jnp.int32)
    dst = e_feat[1].astype(jnp.int32)
    code = (e_attr[:, 0] * 32 + e_attr[:, 1]).astype(jnp.int32)
    loop = jnp.arange(N, dtype=jnp.int32)
    pad = IP - 2 * (E + N)
    half = pad // 2
    xg = jnp.concatenate([src, loop, jnp.zeros((half,), jnp.int32)])
    pg = jnp.concatenate(
        [code, jnp.full((N,), 4 * 32, jnp.int32),
         jnp.zeros((half,), jnp.int32)]
    )
    dd = jnp.concatenate([dst, loop, jnp.full((half,), N, jnp.int32)])
    roll = (IP // 2 // K) // 2
    pgr = jnp.roll(pg.reshape(-1, K), roll, axis=0)
    ddr = jnp.roll(dd.reshape(-1, K), roll, axis=0)
    gl = jnp.stack([xg.reshape(-1, K), pgr], axis=1).reshape(-1)
    dl = jnp.stack([dd.reshape(-1, K), ddr], axis=1).reshape(-1)
    pair = (emb_table[:, None, :] + emb_table[None, :, :]).reshape(1024, D)
    out = _propagate(x, pair, gl, dl)
    return out[:N]


# speedup vs baseline: 1.1646x; 1.1646x over previous
import functools

import jax
import jax.numpy as jnp
from jax import lax
from jax.experimental import pallas as pl
from jax.experimental.pallas import tpu as pltpu
from jax.experimental.pallas import tpu_sc as plsc

N = 10000
D = 128
E = 320000
NC = 2
NS = 16
K = 64
NBUF = 4
CHUNKS = 324
C0 = 368
C1 = 2 * CHUNKS - C0
IPS = K * CHUNKS
IPC = IPS * NS
IP = IPC * NC
OUT_ROWS = 10240
ZROWS = 32
WB = OUT_ROWS // NS


def _sc_kernel(x, pair, gl, dl, out, out_sh, pair_sh, ibufs, xbufs, zbuf,
               gsems, ssems):
    c = lax.axis_index("c")
    s = lax.axis_index("s")

    z = jnp.zeros((16,), jnp.float32)

    def zero_row(i, carry):
        for k in range(D // 16):
            zbuf[i, pl.ds(k * 16, 16)] = z
        return carry

    lax.fori_loop(0, ZROWS, zero_row, 0)

    slab = s * WB

    def zero_slab(j, carry):
        pltpu.sync_copy(zbuf, out_sh.at[pl.ds(slab + j * ZROWS, ZROWS)])
        return carry

    lax.fori_loop(0, WB // ZROWS, zero_slab, 0)

    pltpu.sync_copy(pair.at[pl.ds(s * 64, 64)], pair_sh.at[pl.ds(s * 64, 64)])

    plsc.subcore_barrier()

    nch = jnp.where(c == 0, C0, C1)
    first = jnp.where(c == 0, s * (C0 * K), NS * (C0 * K) + s * (C1 * K))

    def load_idx(g, b):
        base = first + g * K
        pltpu.sync_copy(gl.at[pl.ds(base, K)], ibufs.at[b, 0])
        pltpu.sync_copy(dl.at[pl.ds(base, K)], ibufs.at[b, 1])

    def gather(b):
        srcref = x if b % 2 == 0 else pair_sh
        pltpu.async_copy(srcref.at[ibufs.at[b, 0]], xbufs.at[b], gsems.at[b])

    def wait_gather(b):
        srcref = x if b % 2 == 0 else pair_sh
        pltpu.make_async_copy(
            srcref.at[ibufs.at[b, 0]], xbufs.at[b], gsems.at[b]).wait()

    def scatter(b):
        pltpu.async_copy(
            xbufs.at[b], out_sh.at[ibufs.at[b, 1]], ssems.at[b], add=True)

    def wait_scatter(b):
        pltpu.make_async_copy(
            xbufs.at[b], out_sh.at[ibufs.at[b, 1]], ssems.at[b]).wait()

    for b in range(NBUF - 1):
        load_idx(b, b)
        gather(b)

    def step(t, carry):
        for b in range(NBUF):
            g = NBUF * t + b
            wait_gather(b)
            scatter(b)
            nb = (b + NBUF - 1) % NBUF

            @pl.when(g >= 1)
            def _():
                wait_scatter(nb)

            @pl.when(g + NBUF - 1 < nch)
            def _():
                load_idx(g + NBUF - 1, nb)
                gather(nb)
        return carry

    lax.fori_loop(0, nch // NBUF, step, 0)
    wait_scatter(NBUF - 1)

    plsc.subcore_barrier()

    pltpu.sync_copy(out_sh.at[pl.ds(slab, WB)], out.at[c].at[pl.ds(slab, WB)])


def _merge_kernel(p_ref, o_ref):
    o_ref[...] = p_ref[0] + p_ref[1]


@jax.jit
def _propagate(x, pair, gl, dl):
    mesh = plsc.VectorSubcoreMesh(core_axis_name="c", subcore_axis_name="s")
    partials = pl.kernel(
        _sc_kernel,
        out_type=jax.ShapeDtypeStruct((NC, OUT_ROWS, D), jnp.float32),
        mesh=mesh,
        scratch_types=[
            pltpu.VMEM_SHARED((OUT_ROWS, D), jnp.float32),
            pltpu.VMEM_SHARED((1024, D), jnp.float32),
            pltpu.VMEM((NBUF, 2, K), jnp.int32),
            pltpu.VMEM((NBUF, K, D), jnp.float32),
            pltpu.VMEM((ZROWS, D), jnp.float32),
            pltpu.SemaphoreType.DMA((NBUF,)),
            pltpu.SemaphoreType.DMA((NBUF,)),
        ],
    )(x, pair, gl, dl)
    blk = 1024
    merged = pl.pallas_call(
        _merge_kernel,
        grid=(OUT_ROWS // blk,),
        in_specs=[pl.BlockSpec((NC, blk, D), lambda i: (0, i, 0))],
        out_specs=pl.BlockSpec((blk, D), lambda i: (i, 0)),
        out_shape=jax.ShapeDtypeStruct((OUT_ROWS, D), jnp.float32),
    )(partials)
    return merged


def kernel(x, e_feat, e_attr, emb_table):
    src = e_feat[0].astype(jnp.int32)
    dst = e_feat[1].astype(jnp.int32)
    code = (e_attr[:, 0] * 32 + e_attr[:, 1]).astype(jnp.int32)
    loop = jnp.arange(N, dtype=jnp.int32)
    pad = IP - 2 * (E + N)
    half = pad // 2
    xg = jnp.concatenate([src, loop, jnp.zeros((half,), jnp.int32)])
    pg = jnp.concatenate(
        [code, jnp.full((N,), 4 * 32, jnp.int32),
         jnp.zeros((half,), jnp.int32)]
    )
    dd = jnp.concatenate([dst, loop, jnp.full((half,), N, jnp.int32)])
    roll = (IP // 2 // K) // 2
    pgr = jnp.roll(pg.reshape(-1, K), roll, axis=0)
    ddr = jnp.roll(dd.reshape(-1, K), roll, axis=0)
    gl = jnp.stack([xg.reshape(-1, K), pgr], axis=1).reshape(-1)
    dl = jnp.stack([dd.reshape(-1, K), ddr], axis=1).reshape(-1)
    pair = (emb_table[:, None, :] + emb_table[None, :, :]).reshape(1024, D)
    out = _propagate(x, pair, gl, dl)
    return out[:N]
